# baseline (speedup 1.0000x reference)
import functools

import jax
import jax.numpy as jnp
from jax import lax
from jax.experimental import pallas as pl
from jax.experimental.pallas import tpu as pltpu
from jax.experimental.pallas import tpu_sc as plsc

N = 10000
E = 160000
D = 256
NQ = 4
Q = D // NQ
NC = 2
NS = 16
NW = NC * NS
NP = 10240
RPW = NP // NS

KD = 40
CHD = E // (NW * KD)

KM = 80
CHM = E // (NS * KM)
NBUF = 4
CHM_UP = (CHM + NBUF - 1) // NBUF * NBUF

def _sc_degrees_body(ei_hbm, z16_hbm, out_hbm, idx_v, ones_v, acc_sh):
    c = lax.axis_index("c")
    s = lax.axis_index("s")
    wid = c * NS + s
    pltpu.sync_copy(ei_hbm.at[0].at[wid], idx_v)

    @pl.loop(0, KD)
    def _fill(i):
        ones_v[i, :] = jnp.full((16,), 1.0, jnp.float32)

    pltpu.sync_copy(z16_hbm.at[pl.ds(s * RPW, RPW)],
                    acc_sh.at[pl.ds(s * RPW, RPW)])
    plsc.subcore_barrier()

    @pl.loop(0, CHD)
    def _scatter(j):
        pltpu.sync_copy(ones_v, acc_sh.at[idx_v.at[j]], add=True)

    plsc.subcore_barrier()
    pltpu.sync_copy(acc_sh.at[pl.ds(s * RPW, RPW)],
                    out_hbm.at[pl.ds(s * RPW, RPW), pl.ds(c * 16, 16)])


def _sc_aggregate_body(ei_hbm, y_hbm, z_hbm, out_hbm,
                       ridx, cidx, rows, acc_sh, semg, sems):
    c = lax.axis_index("c")
    s = lax.axis_index("s")
    pltpu.sync_copy(ei_hbm.at[0].at[s], ridx)
    pltpu.sync_copy(ei_hbm.at[1].at[s], cidx)

    for p in range(2):
        q = c * 2 + p
        if p == 0:
            dvec = jnp.broadcast_to((c * 2).astype(jnp.int32), (16,))

            @pl.loop(0, CHM)
            def _shift(i):
                for k in range(KM // 16):
                    sl = pl.ds(k * 16, 16)
                    cidx[i, sl] = (cidx[i, sl] << 2) + dvec
        else:
            dvec = jnp.full((16,), 1, jnp.int32)

            @pl.loop(0, CHM)
            def _shift(i):
                for k in range(KM // 16):
                    sl = pl.ds(k * 16, 16)
                    cidx[i, sl] = cidx[i, sl] + dvec

        pltpu.sync_copy(z_hbm.at[pl.ds(s * RPW, RPW)],
                        acc_sh.at[pl.ds(s * RPW, RPW)])
        plsc.subcore_barrier()

        for b in range(2):
            pltpu.async_copy(y_hbm.at[cidx.at[b]], rows.at[b], semg.at[b])

        @pl.loop(0, CHM_UP, step=NBUF)
        def _agg(j):
            for b in range(NBUF):
                jj = j + b
                bf = (b + 2) % NBUF

                @pl.when(jnp.logical_and(jj >= 2, jj <= CHM + 1))
                def _():
                    pltpu.make_async_copy(
                        rows.at[bf], acc_sh.at[ridx.at[jj - 2]],
                        sems.at[bf]).wait()

                @pl.when(jj + 2 < CHM)
                def _():
                    pltpu.async_copy(y_hbm.at[cidx.at[jj + 2]],
                                     rows.at[bf], semg.at[bf])

                @pl.when(jj < CHM)
                def _():
                    pltpu.make_async_copy(
                        y_hbm.at[cidx.at[jj]], rows.at[b], semg.at[b]).wait()
                    pltpu.async_copy(rows.at[b], acc_sh.at[ridx.at[jj]],
                                     sems.at[b], add=True)

        plsc.subcore_barrier()
        pltpu.sync_copy(acc_sh.at[pl.ds(s * RPW, RPW)],
                        out_hbm.at[pl.ds(c * NP + s * RPW, RPW),
                                   pl.ds(p * Q, Q)])


@functools.cache
def _build_sc_kernels():
    mesh = plsc.VectorSubcoreMesh(
        core_axis_name="c", subcore_axis_name="s",
        num_cores=NC, num_subcores=NS)
    sc_degrees = pl.kernel(
        _sc_degrees_body,
        out_type=jax.ShapeDtypeStruct((NP, 128), jnp.float32),
        mesh=mesh,
        scratch_types=[
            pltpu.VMEM((CHD, KD), jnp.int32),
            pltpu.VMEM((KD, 16), jnp.float32),
            pltpu.VMEM_SHARED((NP, 16), jnp.float32),
        ],
        compiler_params=pltpu.CompilerParams(use_tc_tiling_on_sc=False),
    )
    sc_aggregate = pl.kernel(
        _sc_aggregate_body,
        out_type=jax.ShapeDtypeStruct((NC * NP, 128), jnp.float32),
        mesh=mesh,
        scratch_types=[
            pltpu.VMEM((CHM, KM), jnp.int32),
            pltpu.VMEM((CHM, KM), jnp.int32),
            pltpu.VMEM((NBUF, KM, Q), jnp.float32),
            pltpu.VMEM_SHARED((NP, Q), jnp.float32),
            pltpu.SemaphoreType.DMA((NBUF,)),
            pltpu.SemaphoreType.DMA((NBUF,)),
        ],
        compiler_params=pltpu.CompilerParams(use_tc_tiling_on_sc=False),
    )
    return sc_degrees, sc_aggregate


_BR = 2000


def _dis_block(d_ref):
    deg = d_ref[:, 0:1] + d_ref[:, 16:17]
    return jnp.where(deg > 0.0, lax.rsqrt(deg), 0.0)


def _matmul_body(x_ref, w_ref, y_ref):
    y_ref[...] = jnp.dot(x_ref[...], w_ref[...],
                         preferred_element_type=jnp.float32)


def _tc_matmul(x, w):
    return pl.pallas_call(
        _matmul_body,
        grid=(N // _BR,),
        in_specs=[
            pl.BlockSpec((_BR, D), lambda i: (i, 0)),
            pl.BlockSpec((D, D), lambda i: (0, 0)),
        ],
        out_specs=pl.BlockSpec((_BR, D), lambda i: (i, 0)),
        out_shape=jax.ShapeDtypeStruct((N, D), jnp.float32),
    )(x, w)


def _pack_body(y_ref, d_ref, o_ref):
    y = y_ref[...] * _dis_block(d_ref)
    o_ref[...] = y.reshape(2 * _BR, 128)


def _tc_pack(y, deg2):
    return pl.pallas_call(
        _pack_body,
        grid=(N // _BR,),
        in_specs=[
            pl.BlockSpec((_BR, D), lambda i: (i, 0)),
            pl.BlockSpec((_BR, 128), lambda i: (i, 0)),
        ],
        out_specs=pl.BlockSpec((2 * _BR, 128), lambda i: (i, 0)),
        out_shape=jax.ShapeDtypeStruct((2 * N, 128), jnp.float32),
    )(y, deg2)


def _final_body(s_ref, d_ref, b_ref, o_ref):
    dis = _dis_block(d_ref)
    h = jnp.concatenate([s_ref[0], s_ref[1]], axis=1)
    o_ref[...] = jnp.maximum(h * dis + b_ref[...], 0.0)


def _tc_final(s_parts, deg2, bias):
    return pl.pallas_call(
        _final_body,
        grid=(N // _BR,),
        in_specs=[
            pl.BlockSpec((NC, _BR, 128), lambda i: (0, i, 0)),
            pl.BlockSpec((_BR, 128), lambda i: (i, 0)),
            pl.BlockSpec((1, D), lambda i: (0, 0)),
        ],
        out_specs=pl.BlockSpec((_BR, D), lambda i: (i, 0)),
        out_shape=jax.ShapeDtypeStruct((N, D), jnp.float32),
    )(s_parts, deg2, bias)


def kernel(features, edge_index, weight, bias):
    features = features.astype(jnp.float32)
    ei = edge_index.astype(jnp.int32)

    sc_degrees, sc_aggregate = _build_sc_kernels()
    deg2 = sc_degrees(ei.reshape(2, NW, CHD, KD),
                      jnp.zeros((NP, 16), jnp.float32))

    y = _tc_matmul(features, weight)
    ystack = _tc_pack(y, deg2).reshape(NQ * N, Q)

    s_parts = sc_aggregate(
        ei.reshape(2, NS, CHM, KM),
        ystack,
        jnp.zeros((NP, Q), jnp.float32),
    ).reshape(NC, NP, 128)

    return _tc_final(s_parts, deg2, bias.reshape(1, D))


# speedup vs baseline: 1.0474x; 1.0474x over previous
import functools

import jax
import jax.numpy as jnp
from jax import lax
from jax.experimental import pallas as pl
from jax.experimental.pallas import tpu as pltpu
from jax.experimental.pallas import tpu_sc as plsc

N = 10000
E = 160000
D = 256
NQ = 4
Q = D // NQ
NC = 2
NS = 16
NW = NC * NS
NP = 10240
RPW = NP // NS

KD = 40
CHD = E // (NW * KD)

KM = 80
CHM = E // (NS * KM)
NBUF = 4
CHM_UP = (CHM + NBUF - 1) // NBUF * NBUF

def _sc_degrees_body(ei_hbm, z16_hbm, out_hbm, idx_v, ones_v, acc_sh, semd):
    c = lax.axis_index("c")
    s = lax.axis_index("s")
    wid = c * NS + s
    pltpu.sync_copy(ei_hbm.at[0].at[wid], idx_v)

    @pl.loop(0, KD)
    def _fill(i):
        ones_v[i, :] = jnp.full((16,), 1.0, jnp.float32)

    pltpu.sync_copy(z16_hbm.at[pl.ds(s * RPW, RPW)],
                    acc_sh.at[pl.ds(s * RPW, RPW)])
    plsc.subcore_barrier()

    @pl.loop(0, CHD)
    def _scatter(j):
        pltpu.async_copy(ones_v, acc_sh.at[idx_v.at[j]], semd, add=True)

    @pl.loop(0, CHD)
    def _drain(j):
        pltpu.make_async_copy(ones_v, acc_sh.at[idx_v.at[j]], semd).wait()

    plsc.subcore_barrier()
    pltpu.sync_copy(acc_sh.at[pl.ds(s * RPW, RPW)],
                    out_hbm.at[pl.ds(s * RPW, RPW), pl.ds(c * 16, 16)])


def _sc_aggregate_body(ei_hbm, y_hbm, z_hbm, out_hbm,
                       ridx, cidx, rows, acc_sh, semg, sems):
    c = lax.axis_index("c")
    s = lax.axis_index("s")
    pltpu.sync_copy(ei_hbm.at[0].at[s], ridx)
    pltpu.sync_copy(ei_hbm.at[1].at[s], cidx)

    for p in range(2):
        q = c * 2 + p
        if p == 0:
            dvec = jnp.broadcast_to((c * 2).astype(jnp.int32), (16,))

            @pl.loop(0, CHM)
            def _shift(i):
                for k in range(KM // 16):
                    sl = pl.ds(k * 16, 16)
                    cidx[i, sl] = (cidx[i, sl] << 2) + dvec
        else:
            dvec = jnp.full((16,), 1, jnp.int32)

            @pl.loop(0, CHM)
            def _shift(i):
                for k in range(KM // 16):
                    sl = pl.ds(k * 16, 16)
                    cidx[i, sl] = cidx[i, sl] + dvec

        pltpu.sync_copy(z_hbm.at[pl.ds(s * RPW, RPW)],
                        acc_sh.at[pl.ds(s * RPW, RPW)])
        plsc.subcore_barrier()

        for b in range(2):
            pltpu.async_copy(y_hbm.at[cidx.at[b]], rows.at[b], semg.at[b])

        @pl.loop(0, CHM_UP, step=NBUF)
        def _agg(j):
            for b in range(NBUF):
                jj = j + b
                bf = (b + 2) % NBUF

                @pl.when(jnp.logical_and(jj >= 2, jj <= CHM + 1))
                def _():
                    pltpu.make_async_copy(
                        rows.at[bf], acc_sh.at[ridx.at[jj - 2]],
                        sems.at[bf]).wait()

                @pl.when(jj + 2 < CHM)
                def _():
                    pltpu.async_copy(y_hbm.at[cidx.at[jj + 2]],
                                     rows.at[bf], semg.at[bf])

                @pl.when(jj < CHM)
                def _():
                    pltpu.make_async_copy(
                        y_hbm.at[cidx.at[jj]], rows.at[b], semg.at[b]).wait()
                    pltpu.async_copy(rows.at[b], acc_sh.at[ridx.at[jj]],
                                     sems.at[b], add=True)

        plsc.subcore_barrier()
        pltpu.sync_copy(acc_sh.at[pl.ds(s * RPW, RPW)],
                        out_hbm.at[pl.ds(c * NP + s * RPW, RPW),
                                   pl.ds(p * Q, Q)])


@functools.cache
def _build_sc_kernels():
    mesh = plsc.VectorSubcoreMesh(
        core_axis_name="c", subcore_axis_name="s",
        num_cores=NC, num_subcores=NS)
    sc_degrees = pl.kernel(
        _sc_degrees_body,
        out_type=jax.ShapeDtypeStruct((NP, 128), jnp.float32),
        mesh=mesh,
        scratch_types=[
            pltpu.VMEM((CHD, KD), jnp.int32),
            pltpu.VMEM((KD, 16), jnp.float32),
            pltpu.VMEM_SHARED((NP, 16), jnp.float32),
            pltpu.SemaphoreType.DMA,
        ],
        compiler_params=pltpu.CompilerParams(use_tc_tiling_on_sc=False),
    )
    sc_aggregate = pl.kernel(
        _sc_aggregate_body,
        out_type=jax.ShapeDtypeStruct((NC * NP, 128), jnp.float32),
        mesh=mesh,
        scratch_types=[
            pltpu.VMEM((CHM, KM), jnp.int32),
            pltpu.VMEM((CHM, KM), jnp.int32),
            pltpu.VMEM((NBUF, KM, Q), jnp.float32),
            pltpu.VMEM_SHARED((NP, Q), jnp.float32),
            pltpu.SemaphoreType.DMA((NBUF,)),
            pltpu.SemaphoreType.DMA((NBUF,)),
        ],
        compiler_params=pltpu.CompilerParams(use_tc_tiling_on_sc=False),
    )
    return sc_degrees, sc_aggregate


_BR = 2000


def _dis_block(d_ref):
    deg = d_ref[:, 0:1] + d_ref[:, 16:17]
    return jnp.where(deg > 0.0, lax.rsqrt(deg), 0.0)


def _matmul_body(x_ref, w_ref, y_ref):
    y_ref[...] = jnp.dot(x_ref[...], w_ref[...],
                         preferred_element_type=jnp.float32)


def _tc_matmul(x, w):
    return pl.pallas_call(
        _matmul_body,
        grid=(N // _BR,),
        in_specs=[
            pl.BlockSpec((_BR, D), lambda i: (i, 0)),
            pl.BlockSpec((D, D), lambda i: (0, 0)),
        ],
        out_specs=pl.BlockSpec((_BR, D), lambda i: (i, 0)),
        out_shape=jax.ShapeDtypeStruct((N, D), jnp.float32),
    )(x, w)


def _pack_body(y_ref, d_ref, o_ref):
    y = y_ref[...] * _dis_block(d_ref)
    o_ref[...] = y.reshape(2 * _BR, 128)


def _tc_pack(y, deg2):
    return pl.pallas_call(
        _pack_body,
        grid=(N // _BR,),
        in_specs=[
            pl.BlockSpec((_BR, D), lambda i: (i, 0)),
            pl.BlockSpec((_BR, 128), lambda i: (i, 0)),
        ],
        out_specs=pl.BlockSpec((2 * _BR, 128), lambda i: (i, 0)),
        out_shape=jax.ShapeDtypeStruct((2 * N, 128), jnp.float32),
    )(y, deg2)


def _final_body(s_ref, d_ref, b_ref, o_ref):
    dis = _dis_block(d_ref)
    h = jnp.concatenate([s_ref[0], s_ref[1]], axis=1)
    o_ref[...] = jnp.maximum(h * dis + b_ref[...], 0.0)


def _tc_final(s_parts, deg2, bias):
    return pl.pallas_call(
        _final_body,
        grid=(N // _BR,),
        in_specs=[
            pl.BlockSpec((NC, _BR, 128), lambda i: (0, i, 0)),
            pl.BlockSpec((_BR, 128), lambda i: (i, 0)),
            pl.BlockSpec((1, D), lambda i: (0, 0)),
        ],
        out_specs=pl.BlockSpec((_BR, D), lambda i: (i, 0)),
        out_shape=jax.ShapeDtypeStruct((N, D), jnp.float32),
    )(s_parts, deg2, bias)


def kernel(features, edge_index, weight, bias):
    features = features.astype(jnp.float32)
    ei = edge_index.astype(jnp.int32)

    sc_degrees, sc_aggregate = _build_sc_kernels()
    y = _tc_matmul(features, weight)
    deg2 = sc_degrees(ei.reshape(2, NW, CHD, KD),
                      jnp.zeros((NP, 16), jnp.float32))

    ystack = _tc_pack(y, deg2).reshape(NQ * N, Q)

    s_parts = sc_aggregate(
        ei.reshape(2, NS, CHM, KM),
        ystack,
        jnp.zeros((NP, Q), jnp.float32),
    ).reshape(NC, NP, 128)

    return _tc_final(s_parts, deg2, bias.reshape(1, D))
